# baseline (speedup 1.0000x reference)
import functools

import jax
import jax.numpy as jnp
from jax.experimental import pallas as pl

_NUM_LAYERS = 2
_LAM = 0.5
_BETA = 0.25
_SAVE_NUM = 4
_SENT = 1 << 20


def _rup(x, m):
    return ((x + m - 1) // m) * m


def _gather_body(idx_ref, tab_ref, out_ref, *, nb):
    e, b = pl.program_id(0), pl.program_id(1)
    del e
    idx = idx_ref[0, 0, :]
    ids = b * nb + jax.lax.broadcasted_iota(jnp.int32, (nb, 1), 0)[:, 0]
    mask = (ids[:, None] == idx[None, :]).astype(jnp.float32)

    @pl.when(b == 0)
    def _():
        out_ref[...] = jnp.zeros_like(out_ref)

    out_ref[...] += jnp.dot(tab_ref[...], mask,
                            preferred_element_type=jnp.float32)


def _gather(idx3, table, eb, nb):
    ne = idx3.shape[0]
    npad = table.shape[1]
    nnb = npad // nb
    return pl.pallas_call(
        functools.partial(_gather_body, nb=nb),
        grid=(ne, nnb),
        in_specs=[
            pl.BlockSpec((1, 1, eb), lambda e, b: (e, 0, 0)),
            pl.BlockSpec((8, nb), lambda e, b: (0, b)),
        ],
        out_specs=pl.BlockSpec((8, eb), lambda e, b: (0, e)),
        out_shape=jax.ShapeDtypeStruct((8, ne * eb), jnp.float32),
    )(idx3, table)


def _segsum_body(idx_ref, val_ref, out_ref, *, nb):
    b, e = pl.program_id(0), pl.program_id(1)
    idx = idx_ref[0, 0, :]
    ids = b * nb + jax.lax.broadcasted_iota(jnp.int32, (nb, 1), 0)[:, 0]
    mask = (idx[:, None] == ids[None, :]).astype(jnp.float32)

    @pl.when(e == 0)
    def _():
        out_ref[...] = jnp.zeros_like(out_ref)

    out_ref[...] += jnp.dot(val_ref[...], mask,
                            preferred_element_type=jnp.float32)


def _segsum(idx3, vals, npad, eb, nb):
    ne = idx3.shape[0]
    nnb = npad // nb
    return pl.pallas_call(
        functools.partial(_segsum_body, nb=nb),
        grid=(nnb, ne),
        in_specs=[
            pl.BlockSpec((1, 1, eb), lambda b, e: (e, 0, 0)),
            pl.BlockSpec((8, eb), lambda b, e: (0, e)),
        ],
        out_specs=pl.BlockSpec((8, nb), lambda b, e: (0, b)),
        out_shape=jax.ShapeDtypeStruct((8, npad), jnp.float32),
    )(idx3, vals)


def _select_body(gi_ref, si_ref, ki_ref, fi_ref, gj_ref, sj_ref, out_ref,
                 *, nj):
    j = pl.program_id(1)
    gi = gi_ref[0, 0, :]
    si = si_ref[0, 0, :]
    gj = gj_ref[0, 0, :]
    sj = sj_ref[0, 0, :]

    @pl.when(j == 0)
    def _():
        out_ref[...] = jnp.zeros_like(out_ref)

    same = gi[:, None] == gj[None, :]
    gt = sj[None, :] > si[:, None]
    cnt = jnp.sum(jnp.where(same & gt, 1.0, 0.0), axis=1)
    out_ref[0, 0, :] += cnt

    @pl.when(j == nj - 1)
    def _():
        sel = (out_ref[0, 0, :] < ki_ref[0, 0, :]) | (fi_ref[0, 0, :] > 0.5)
        out_ref[0, 0, :] = jnp.where(sel, si, 0.0)


def _select(grp3f, score3, k3, self3, eb):
    ne = grp3f.shape[0]
    espec = pl.BlockSpec((1, 1, eb), lambda i, j: (i, 0, 0))
    jspec = pl.BlockSpec((1, 1, eb), lambda i, j: (j, 0, 0))
    return pl.pallas_call(
        functools.partial(_select_body, nj=ne),
        grid=(ne, ne),
        in_specs=[espec, espec, espec, espec, jspec, jspec],
        out_specs=pl.BlockSpec((1, 1, eb), lambda i, j: (i, 0, 0)),
        out_shape=jax.ShapeDtypeStruct((ne, 1, eb), jnp.float32),
    )(grp3f, score3, k3, self3, grp3f, score3)


def kernel(gumbel, edge_index, attrs):
    src = edge_index[0]
    dst = edge_index[1]
    n = attrs.shape[0]
    e = src.shape[0]

    eb = min(1024, _rup(e, 128))
    nb = min(1024, _rup(n, 128))
    epad = _rup(e, eb)
    npad = _rup(n, nb)
    npad2 = _rup(2 * n, nb)
    ne = epad // eb

    pad_e = epad - e
    src_p = jnp.concatenate([src, jnp.full((pad_e,), _SENT, jnp.int32)])
    dst_p = jnp.concatenate([dst, jnp.full((pad_e,), _SENT, jnp.int32)])
    gum_p = jnp.concatenate([gumbel, jnp.zeros((pad_e,), jnp.float32)])
    src3 = src_p.reshape(ne, 1, eb)
    dst3 = dst_p.reshape(ne, 1, eb)

    valid = jnp.arange(epad, dtype=jnp.int32) < e
    is_self = (src_p == dst_p) & valid
    tril = src_p > dst_p
    triu = (src_p < dst_p) & valid

    delta = (2 * attrs - 1).astype(jnp.float32)
    attrs_f = attrs.astype(jnp.float32)

    def table(rows, width):
        t = jnp.zeros((8, width), jnp.float32)
        for i, r in enumerate(rows):
            t = t.at[i, : r.shape[0]].set(r)
        return t

    def rows2(x):
        z = jnp.zeros((8, epad), jnp.float32)
        for i, r in enumerate(x):
            z = z.at[i, :].set(r)
        return z

    g1 = _gather(src3, table([delta, attrs_f], npad), eb, nb)
    d_src = g1[0]
    a_src = g1[1]

    c0 = jnp.where(tril, d_src, 0.0)
    c1 = jnp.where(triu, d_src, 0.0)
    s1 = _segsum(dst3, rows2([c0, c1]), npad, eb, nb)
    sp0 = delta + s1[0, :n]
    sp1 = delta + s1[1, :n]

    g2 = _gather(src3, table([sp0, sp1], npad), eb, nb)
    c0b = jnp.where(tril, d_src + g2[0], 0.0)
    c1b = jnp.where(triu, d_src + g2[1], 0.0)
    s2 = _segsum(dst3, rows2([c0b, c1b]), npad, eb, nb)
    sp0 = delta + s2[0, :n]
    sp1 = delta + s2[1, :n]

    sp = 1.0 / (jnp.abs(sp0 + sp1) + 1.0)

    g3 = _gather(src3, table([sp], npad), eb, nb)
    w = jnp.where(is_self | ~valid, 0.0, g3[0])

    grp = jnp.where(valid, dst_p * 2 + a_src.astype(jnp.int32), 2 * _SENT)
    grp3 = grp.reshape(ne, 1, eb)

    s3 = _segsum(grp3, rows2([w, (valid & ~is_self).astype(jnp.float32)]),
                 npad2, eb, nb)
    denom = s3[0, : 2 * n]
    cnt = s3[1, : 2 * n].astype(jnp.int32).reshape(n, 2)

    num0 = cnt[:, 0]
    num1 = cnt[:, 1]
    s = attrs
    num0_slf = num0 + (1 - s)
    num1_slf = num1 + s
    k_same0 = jnp.minimum(
        jnp.maximum(_SAVE_NUM,
                    ((num0 + 3).astype(jnp.float32) * _BETA).astype(jnp.int32)),
        num0)
    k_same1 = jnp.minimum(
        jnp.maximum(_SAVE_NUM,
                    ((num1 + 3).astype(jnp.float32) * _BETA).astype(jnp.int32)),
        num1)
    max_num = jnp.minimum(num0.astype(jnp.float32) / (_LAM + 1e-4),
                          num1.astype(jnp.float32) / (1.0001 - _LAM))
    min0 = jnp.round(max_num * _LAM).astype(jnp.int32) - (1 - s)
    min1 = jnp.round(max_num * (1.0 - _LAM)).astype(jnp.int32) - s
    k0_mix = jnp.where(min0 <= 0, 0, jnp.minimum(min0, num0))
    k1_mix = jnp.where(min1 <= 0, 0, jnp.minimum(min1, num1))
    both_zero = (num0 == 0) & (num1 == 0)
    k0 = jnp.where(both_zero, 0,
                   jnp.where(num0_slf == 0, 0,
                             jnp.where(num1_slf == 0, k_same0, k0_mix)))
    k1 = jnp.where(both_zero, 0,
                   jnp.where(num0_slf == 0, k_same1,
                             jnp.where(num1_slf == 0, 0, k1_mix)))
    kgrp = jnp.stack([k0, k1], axis=1).reshape(2 * n).astype(jnp.float32)

    g4 = _gather(grp3, table([denom, kgrp], npad2), eb, nb)
    denom_e = g4[0]
    k_e = g4[1]

    p = jnp.where(is_self, 0.0, w / jnp.maximum(denom_e, 1e-12))
    score = jnp.log(jnp.maximum(p, 1e-20)) + gum_p

    out3 = _select(grp.astype(jnp.float32).reshape(ne, 1, eb),
                   score.reshape(ne, 1, eb),
                   k_e.reshape(ne, 1, eb),
                   is_self.astype(jnp.float32).reshape(ne, 1, eb),
                   eb)
    return out3.reshape(epad)[:e]


# speedup vs baseline: 1.3200x; 1.3200x over previous
import functools

import jax
import jax.numpy as jnp
from jax.experimental import pallas as pl

_NUM_LAYERS = 2
_LAM = 0.5
_BETA = 0.25
_SAVE_NUM = 4
_SENT = 1 << 20


def _rup(x, m):
    return ((x + m - 1) // m) * m


def _gather_body(idx_ref, tab_ref, out_ref, *, nb):
    e, b = pl.program_id(0), pl.program_id(1)
    del e
    idx = idx_ref[0, 0, :]
    ids = b * nb + jax.lax.broadcasted_iota(jnp.int32, (nb, 1), 0)[:, 0]
    mask = (ids[:, None] == idx[None, :]).astype(jnp.float32)

    @pl.when(b == 0)
    def _():
        out_ref[...] = jnp.zeros_like(out_ref)

    out_ref[...] += jnp.dot(tab_ref[...], mask,
                            preferred_element_type=jnp.float32)


def _gather(idx3, table, eb, nb):
    ne = idx3.shape[0]
    npad = table.shape[1]
    nnb = npad // nb
    return pl.pallas_call(
        functools.partial(_gather_body, nb=nb),
        grid=(ne, nnb),
        in_specs=[
            pl.BlockSpec((1, 1, eb), lambda e, b: (e, 0, 0)),
            pl.BlockSpec((8, nb), lambda e, b: (0, b)),
        ],
        out_specs=pl.BlockSpec((8, eb), lambda e, b: (0, e)),
        out_shape=jax.ShapeDtypeStruct((8, ne * eb), jnp.float32),
    )(idx3, table)


def _segsum_body(idx_ref, val_ref, out_ref, *, nb):
    b, e = pl.program_id(0), pl.program_id(1)
    idx = idx_ref[0, 0, :]
    ids = b * nb + jax.lax.broadcasted_iota(jnp.int32, (nb, 1), 0)[:, 0]
    mask = (idx[:, None] == ids[None, :]).astype(jnp.float32)

    @pl.when(e == 0)
    def _():
        out_ref[...] = jnp.zeros_like(out_ref)

    out_ref[...] += jnp.dot(val_ref[...], mask,
                            preferred_element_type=jnp.float32)


def _segsum(idx3, vals, npad, eb, nb):
    ne = idx3.shape[0]
    nnb = npad // nb
    return pl.pallas_call(
        functools.partial(_segsum_body, nb=nb),
        grid=(nnb, ne),
        in_specs=[
            pl.BlockSpec((1, 1, eb), lambda b, e: (e, 0, 0)),
            pl.BlockSpec((8, eb), lambda b, e: (0, e)),
        ],
        out_specs=pl.BlockSpec((8, nb), lambda b, e: (0, b)),
        out_shape=jax.ShapeDtypeStruct((8, npad), jnp.float32),
    )(idx3, vals)


def _select_body(gi_ref, si_ref, ki_ref, fi_ref, gj_ref, sj_ref, out_ref,
                 *, nj):
    j = pl.program_id(1)
    gi = gi_ref[0, 0, :]
    si = si_ref[0, 0, :]
    gj = gj_ref[0, 0, :]
    sj = sj_ref[0, 0, :]

    @pl.when(j == 0)
    def _():
        out_ref[...] = jnp.zeros_like(out_ref)

    same = gi[:, None] == gj[None, :]
    gt = sj[None, :] > si[:, None]
    cnt = jnp.sum(jnp.where(same & gt, 1.0, 0.0), axis=1)
    out_ref[0, 0, :] += cnt

    @pl.when(j == nj - 1)
    def _():
        sel = (out_ref[0, 0, :] < ki_ref[0, 0, :]) | (fi_ref[0, 0, :] > 0.5)
        out_ref[0, 0, :] = jnp.where(sel, si, 0.0)


def _select(grp3f, score3, k3, self3, eb):
    ne = grp3f.shape[0]
    espec = pl.BlockSpec((1, 1, eb), lambda i, j: (i, 0, 0))
    jspec = pl.BlockSpec((1, 1, eb), lambda i, j: (j, 0, 0))
    return pl.pallas_call(
        functools.partial(_select_body, nj=ne),
        grid=(ne, ne),
        in_specs=[espec, espec, espec, espec, jspec, jspec],
        out_specs=pl.BlockSpec((1, 1, eb), lambda i, j: (i, 0, 0)),
        out_shape=jax.ShapeDtypeStruct((ne, 1, eb), jnp.float32),
    )(grp3f, score3, k3, self3, grp3f, score3)


def kernel(gumbel, edge_index, attrs):
    src = edge_index[0]
    dst = edge_index[1]
    n = attrs.shape[0]
    e = src.shape[0]

    eb = min(1024, _rup(e, 128))
    nb = min(1024, _rup(n, 128))
    epad = _rup(e, eb)
    npad = _rup(n, nb)
    npad2 = _rup(2 * n, nb)
    ne = epad // eb

    pad_e = epad - e
    src_p = jnp.concatenate([src, jnp.full((pad_e,), _SENT, jnp.int32)])
    dst_p = jnp.concatenate([dst, jnp.full((pad_e,), _SENT, jnp.int32)])
    gum_p = jnp.concatenate([gumbel, jnp.zeros((pad_e,), jnp.float32)])
    src3 = src_p.reshape(ne, 1, eb)
    dst3 = dst_p.reshape(ne, 1, eb)

    valid = jnp.arange(epad, dtype=jnp.int32) < e
    is_self = (src_p == dst_p) & valid
    tril = src_p > dst_p
    triu = (src_p < dst_p) & valid

    delta = (2 * attrs - 1).astype(jnp.float32)
    attrs_f = attrs.astype(jnp.float32)

    def table(rows, width):
        t = jnp.zeros((8, width), jnp.float32)
        for i, r in enumerate(rows):
            t = t.at[i, : r.shape[0]].set(r)
        return t

    def rows2(x):
        z = jnp.zeros((8, epad), jnp.float32)
        for i, r in enumerate(x):
            z = z.at[i, :].set(r)
        return z

    g1 = _gather(src3, table([delta, attrs_f], npad), eb, nb)
    d_src = g1[0]
    a_src = g1[1]

    c0 = jnp.where(tril, d_src, 0.0)
    c1 = jnp.where(triu, d_src, 0.0)
    s1 = _segsum(dst3, rows2([c0, c1]), npad, eb, nb)
    sp0 = delta + s1[0, :n]
    sp1 = delta + s1[1, :n]

    g2 = _gather(src3, table([sp0, sp1], npad), eb, nb)
    c0b = jnp.where(tril, d_src + g2[0], 0.0)
    c1b = jnp.where(triu, d_src + g2[1], 0.0)
    s2 = _segsum(dst3, rows2([c0b, c1b]), npad, eb, nb)
    sp0 = delta + s2[0, :n]
    sp1 = delta + s2[1, :n]

    sp = 1.0 / (jnp.abs(sp0 + sp1) + 1.0)

    g3 = _gather(src3, table([sp], npad), eb, nb)
    w = jnp.where(is_self | ~valid, 0.0, g3[0])

    grp = jnp.where(valid, dst_p * 2 + a_src.astype(jnp.int32), 2 * _SENT)
    grp3 = grp.reshape(ne, 1, eb)

    s3 = _segsum(grp3, rows2([w, (valid & ~is_self).astype(jnp.float32)]),
                 npad2, eb, nb)
    denom = s3[0, : 2 * n]
    cnt = s3[1, : 2 * n].astype(jnp.int32).reshape(n, 2)

    num0 = cnt[:, 0]
    num1 = cnt[:, 1]
    s = attrs
    num0_slf = num0 + (1 - s)
    num1_slf = num1 + s
    k_same0 = jnp.minimum(
        jnp.maximum(_SAVE_NUM,
                    ((num0 + 3).astype(jnp.float32) * _BETA).astype(jnp.int32)),
        num0)
    k_same1 = jnp.minimum(
        jnp.maximum(_SAVE_NUM,
                    ((num1 + 3).astype(jnp.float32) * _BETA).astype(jnp.int32)),
        num1)
    max_num = jnp.minimum(num0.astype(jnp.float32) / (_LAM + 1e-4),
                          num1.astype(jnp.float32) / (1.0001 - _LAM))
    min0 = jnp.round(max_num * _LAM).astype(jnp.int32) - (1 - s)
    min1 = jnp.round(max_num * (1.0 - _LAM)).astype(jnp.int32) - s
    k0_mix = jnp.where(min0 <= 0, 0, jnp.minimum(min0, num0))
    k1_mix = jnp.where(min1 <= 0, 0, jnp.minimum(min1, num1))
    both_zero = (num0 == 0) & (num1 == 0)
    k0 = jnp.where(both_zero, 0,
                   jnp.where(num0_slf == 0, 0,
                             jnp.where(num1_slf == 0, k_same0, k0_mix)))
    k1 = jnp.where(both_zero, 0,
                   jnp.where(num0_slf == 0, k_same1,
                             jnp.where(num1_slf == 0, 0, k1_mix)))
    kgrp = jnp.stack([k0, k1], axis=1).reshape(2 * n).astype(jnp.float32)

    g4 = _gather(grp3, table([denom, kgrp], npad2), eb, nb)
    denom_e = g4[0]
    k_e = g4[1]

    p = jnp.where(is_self, 0.0, w / jnp.maximum(denom_e, 1e-12))
    score = jnp.log(jnp.maximum(p, 1e-20)) + gum_p

    sb = min(2048, _rup(e, 128))
    epad_s = _rup(e, sb)
    ns = epad_s // sb

    def spad(x, fill):
        x = x[:e]
        return jnp.concatenate(
            [x, jnp.full((epad_s - e,), fill, x.dtype)]).reshape(ns, 1, sb)

    out3 = _select(spad(grp.astype(jnp.float32), float(2 * _SENT)),
                   spad(score, 0.0),
                   spad(k_e, 0.0),
                   spad(is_self.astype(jnp.float32), 0.0),
                   sb)
    return out3.reshape(epad_s)[:e]


# speedup vs baseline: 1.4667x; 1.1112x over previous
import functools

import jax
import jax.numpy as jnp
from jax.experimental import pallas as pl

_NUM_LAYERS = 2
_LAM = 0.5
_BETA = 0.25
_SAVE_NUM = 4
_SENT = 1 << 20


def _rup(x, m):
    return ((x + m - 1) // m) * m


def _gather_body(idx_ref, tab_ref, out_ref, *, nb):
    e, b = pl.program_id(0), pl.program_id(1)
    del e
    idx = idx_ref[0, 0, :]
    ids = b * nb + jax.lax.broadcasted_iota(jnp.int32, (nb, 1), 0)[:, 0]
    mask = (ids[:, None] == idx[None, :]).astype(jnp.float32)

    @pl.when(b == 0)
    def _():
        out_ref[...] = jnp.zeros_like(out_ref)

    out_ref[...] += jnp.dot(tab_ref[...], mask,
                            preferred_element_type=jnp.float32)


def _gather(idx3, table, eb, nb):
    ne = idx3.shape[0]
    npad = table.shape[1]
    nnb = npad // nb
    return pl.pallas_call(
        functools.partial(_gather_body, nb=nb),
        grid=(ne, nnb),
        in_specs=[
            pl.BlockSpec((1, 1, eb), lambda e, b: (e, 0, 0)),
            pl.BlockSpec((8, nb), lambda e, b: (0, b)),
        ],
        out_specs=pl.BlockSpec((8, eb), lambda e, b: (0, e)),
        out_shape=jax.ShapeDtypeStruct((8, ne * eb), jnp.float32),
    )(idx3, table)


def _segsum_body(idx_ref, val_ref, out_ref, *, nb):
    b, e = pl.program_id(0), pl.program_id(1)
    idx = idx_ref[0, 0, :]
    ids = b * nb + jax.lax.broadcasted_iota(jnp.int32, (nb, 1), 0)[:, 0]
    mask = (idx[:, None] == ids[None, :]).astype(jnp.float32)

    @pl.when(e == 0)
    def _():
        out_ref[...] = jnp.zeros_like(out_ref)

    out_ref[...] += jnp.dot(val_ref[...], mask,
                            preferred_element_type=jnp.float32)


def _segsum(idx3, vals, npad, eb, nb):
    ne = idx3.shape[0]
    nnb = npad // nb
    return pl.pallas_call(
        functools.partial(_segsum_body, nb=nb),
        grid=(nnb, ne),
        in_specs=[
            pl.BlockSpec((1, 1, eb), lambda b, e: (e, 0, 0)),
            pl.BlockSpec((8, eb), lambda b, e: (0, e)),
        ],
        out_specs=pl.BlockSpec((8, nb), lambda b, e: (0, b)),
        out_shape=jax.ShapeDtypeStruct((8, npad), jnp.float32),
    )(idx3, vals)


def _select_body(gi_ref, si_ref, ki_ref, fi_ref, gj_ref, sj_ref, out_ref,
                 *, nj):
    j = pl.program_id(1)
    gi = gi_ref[0, 0, :]
    si = si_ref[0, 0, :]
    gj = gj_ref[0, 0, :]
    sj = sj_ref[0, 0, :]

    @pl.when(j == 0)
    def _():
        out_ref[...] = jnp.zeros_like(out_ref)

    same = gi[:, None] == gj[None, :]
    gt = sj[None, :] > si[:, None]
    cnt = jnp.sum(jnp.where(same & gt, 1.0, 0.0), axis=1)
    out_ref[0, 0, :] += cnt

    @pl.when(j == nj - 1)
    def _():
        sel = (out_ref[0, 0, :] < ki_ref[0, 0, :]) | (fi_ref[0, 0, :] > 0.5)
        out_ref[0, 0, :] = jnp.where(sel, si, 0.0)


def _select(grp3f, score3, k3, self3, eb):
    ne = grp3f.shape[0]
    espec = pl.BlockSpec((1, 1, eb), lambda i, j: (i, 0, 0))
    jspec = pl.BlockSpec((1, 1, eb), lambda i, j: (j, 0, 0))
    return pl.pallas_call(
        functools.partial(_select_body, nj=ne),
        grid=(ne, ne),
        in_specs=[espec, espec, espec, espec, jspec, jspec],
        out_specs=pl.BlockSpec((1, 1, eb), lambda i, j: (i, 0, 0)),
        out_shape=jax.ShapeDtypeStruct((ne, 1, eb), jnp.float32),
    )(grp3f, score3, k3, self3, grp3f, score3)


def kernel(gumbel, edge_index, attrs):
    src = edge_index[0]
    dst = edge_index[1]
    n = attrs.shape[0]
    e = src.shape[0]

    eb = min(1024, _rup(e, 128))
    nb = min(1024, _rup(n, 128))
    epad = _rup(e, eb)
    npad = _rup(n, nb)
    npad2 = _rup(2 * n, nb)
    ne = epad // eb

    pad_e = epad - e
    src_p = jnp.concatenate([src, jnp.full((pad_e,), _SENT, jnp.int32)])
    dst_p = jnp.concatenate([dst, jnp.full((pad_e,), _SENT, jnp.int32)])
    gum_p = jnp.concatenate([gumbel, jnp.zeros((pad_e,), jnp.float32)])
    src3 = src_p.reshape(ne, 1, eb)
    dst3 = dst_p.reshape(ne, 1, eb)

    valid = jnp.arange(epad, dtype=jnp.int32) < e
    is_self = (src_p == dst_p) & valid
    tril = src_p > dst_p
    triu = (src_p < dst_p) & valid

    delta = (2 * attrs - 1).astype(jnp.float32)
    attrs_f = attrs.astype(jnp.float32)

    def table(rows, width):
        t = jnp.zeros((8, width), jnp.float32)
        for i, r in enumerate(rows):
            t = t.at[i, : r.shape[0]].set(r)
        return t

    def rows2(x):
        z = jnp.zeros((8, epad), jnp.float32)
        for i, r in enumerate(x):
            z = z.at[i, :].set(r)
        return z

    g1 = _gather(src3, table([delta, attrs_f], npad), eb, nb)
    d_src = g1[0]
    a_src = g1[1]

    c0 = jnp.where(tril, d_src, 0.0)
    c1 = jnp.where(triu, d_src, 0.0)
    s1 = _segsum(dst3, rows2([c0, c1]), npad, eb, nb)
    sp0 = delta + s1[0, :n]
    sp1 = delta + s1[1, :n]

    g2 = _gather(src3, table([sp0, sp1], npad), eb, nb)
    c0b = jnp.where(tril, d_src + g2[0], 0.0)
    c1b = jnp.where(triu, d_src + g2[1], 0.0)
    s2 = _segsum(dst3, rows2([c0b, c1b]), npad, eb, nb)
    sp0 = delta + s2[0, :n]
    sp1 = delta + s2[1, :n]

    sp = 1.0 / (jnp.abs(sp0 + sp1) + 1.0)

    g3 = _gather(src3, table([sp], npad), eb, nb)
    w = jnp.where(is_self | ~valid, 0.0, g3[0])

    grp = jnp.where(valid, dst_p * 2 + a_src.astype(jnp.int32), 2 * _SENT)
    grp3 = grp.reshape(ne, 1, eb)

    s3 = _segsum(grp3, rows2([w, (valid & ~is_self).astype(jnp.float32)]),
                 npad2, eb, nb)
    denom = s3[0, : 2 * n]
    cnt = s3[1, : 2 * n].astype(jnp.int32).reshape(n, 2)

    num0 = cnt[:, 0]
    num1 = cnt[:, 1]
    s = attrs
    num0_slf = num0 + (1 - s)
    num1_slf = num1 + s
    k_same0 = jnp.minimum(
        jnp.maximum(_SAVE_NUM,
                    ((num0 + 3).astype(jnp.float32) * _BETA).astype(jnp.int32)),
        num0)
    k_same1 = jnp.minimum(
        jnp.maximum(_SAVE_NUM,
                    ((num1 + 3).astype(jnp.float32) * _BETA).astype(jnp.int32)),
        num1)
    max_num = jnp.minimum(num0.astype(jnp.float32) / (_LAM + 1e-4),
                          num1.astype(jnp.float32) / (1.0001 - _LAM))
    min0 = jnp.round(max_num * _LAM).astype(jnp.int32) - (1 - s)
    min1 = jnp.round(max_num * (1.0 - _LAM)).astype(jnp.int32) - s
    k0_mix = jnp.where(min0 <= 0, 0, jnp.minimum(min0, num0))
    k1_mix = jnp.where(min1 <= 0, 0, jnp.minimum(min1, num1))
    both_zero = (num0 == 0) & (num1 == 0)
    k0 = jnp.where(both_zero, 0,
                   jnp.where(num0_slf == 0, 0,
                             jnp.where(num1_slf == 0, k_same0, k0_mix)))
    k1 = jnp.where(both_zero, 0,
                   jnp.where(num0_slf == 0, k_same1,
                             jnp.where(num1_slf == 0, 0, k1_mix)))
    kgrp = jnp.stack([k0, k1], axis=1).reshape(2 * n).astype(jnp.float32)

    g4 = _gather(grp3, table([denom, kgrp], npad2), eb, nb)
    denom_e = g4[0]
    k_e = g4[1]

    p = jnp.where(is_self, 0.0, w / jnp.maximum(denom_e, 1e-12))
    score = jnp.log(jnp.maximum(p, 1e-20)) + gum_p

    sb = min(3072, _rup(e, 128))
    epad_s = _rup(e, sb)
    ns = epad_s // sb

    def spad(x, fill):
        x = x[:e]
        return jnp.concatenate(
            [x, jnp.full((epad_s - e,), fill, x.dtype)]).reshape(ns, 1, sb)

    out3 = _select(spad(grp.astype(jnp.float32), float(2 * _SENT)),
                   spad(score, 0.0),
                   spad(k_e, 0.0),
                   spad(is_self.astype(jnp.float32), 0.0),
                   sb)
    return out3.reshape(epad_s)[:e]
